# baseline (speedup 1.0000x reference)
import functools

import jax
import jax.numpy as jnp
from jax import lax
from jax.experimental import pallas as pl
from jax.experimental.pallas import tpu as pltpu
from jax.experimental.pallas import tpu_sc as plsc

B = 16384
D = 128
K = 5
L = 16
NC = 2
NS = 16
NW = NC * NS
BPW = B // NW
C = 32
NBUF = 4
NCHUNK = BPW // C
NV = D // L
SCORES = K + 1
SW = SCORES * L
SWP = SW + 1

_LOG1P = (2.2159764512252877e-07, 0.9999702572822571, -0.4993339478969574,
          0.327511727809906, -0.22396689653396606, 0.13198965787887573,
          -0.053267478942871094, 0.010243828408420086)


def _log_sigmoid(s):
    y = jnp.exp(-jnp.abs(s))
    p = _LOG1P[7] * y + _LOG1P[6]
    for c in _LOG1P[5::-1]:
        p = p * y + c
    return jnp.minimum(s, 0.0) - p


def _sc_loss_kernel(vi_hbm, vj_hbm, vn_hbm, emb_hbm, ctx_hbm, out_hbm,
                    idx_i, idx_j, idx_n,
                    rows_i0, rows_j0, rows_n0,
                    rows_i1, rows_j1, rows_n1,
                    rows_i2, rows_j2, rows_n2,
                    rows_i3, rows_j3, rows_n3,
                    acc, stage, gbuf, shared, sem0, sem1, sem2, sem3):
    cid = lax.axis_index("c")
    sid = lax.axis_index("s")
    wid = sid * NC + cid
    base = wid * BPW

    pltpu.sync_copy(vi_hbm.at[pl.ds(base, BPW)], idx_i)
    pltpu.sync_copy(vj_hbm.at[pl.ds(base, BPW)], idx_j)
    pltpu.sync_copy(vn_hbm.at[pl.ds(base * K, BPW * K)], idx_n)

    bufs = ((rows_i0, rows_j0, rows_n0, sem0),
            (rows_i1, rows_j1, rows_n1, sem1),
            (rows_i2, rows_j2, rows_n2, sem2),
            (rows_i3, rows_j3, rows_n3, sem3))

    def copies(ci, parity):
        ri, rj, rn, sem = bufs[parity]
        o = ci * C
        cps = [
            (emb_hbm.at[idx_i.at[pl.ds(o, C)]], ri, sem),
            (ctx_hbm.at[idx_j.at[pl.ds(o, C)]], rj, sem),
        ]
        for s, n in ((0, 128), (128, 32)):
            cps.append((ctx_hbm.at[idx_n.at[pl.ds(o * K + s, n)]],
                        rn.at[pl.ds(s, n), :], sem))
        return cps

    def fire(ci, parity):
        for src, dst, sem in copies(ci, parity):
            pltpu.async_copy(src, dst, sem)

    def drain(ci, parity):
        for src, dst, sem in copies(ci, parity):
            pltpu.make_async_copy(src, dst, sem).wait()

    iota16 = lax.iota(jnp.int32, L)
    zeros16 = iota16 * 0

    def compute(parity, tot):
        ri, rj, rn, _ = bufs[parity]

        def tree_dot(u, ref, row):
            ps = [u[l] * ref[row, pl.ds(L * l, L)] for l in range(NV)]
            while len(ps) > 1:
                ps = [a + b for a, b in zip(ps[::2], ps[1::2])]
            return ps[0]

        with jax.named_scope("stageA"):
            @plsc.parallel_loop(0, C, unroll=4)
            def _stage_a(e):
                u = [ri[e, pl.ds(L * l, L)] for l in range(NV)]
                acc[e, pl.ds(0, L)] = tree_dot(u, rj, e)
                for k in range(K):
                    acc[e, pl.ds(L * (k + 1), L)] = tree_dot(u, rn, K * e + k)

        with jax.named_scope("stageB"):
            @plsc.parallel_loop(0, C // L, unroll=2, carry=tot)
            def tot(g, tot2):
                rowidx = iota16 + g * L
                for j in range(SCORES):
                    gs = [plsc.load_gather(acc, [rowidx, zeros16 + (L * j + l)])
                          for l in range(L)]
                    while len(gs) > 1:
                        gs = [a + b for a, b in zip(gs[::2], gs[1::2])]
                    tot2 = tot2 + _log_sigmoid(gs[0] if j == 0 else -gs[0])
                return tot2
        return tot

    for b in range(NBUF - 1):
        fire(b, b)

    def ring_body(p, tot):
        ci = NBUF * p
        for b in range(NBUF):
            with jax.named_scope("drain0"):
                drain(ci + b, b)
            tot = compute(b, tot)

            @pl.when(ci + b + (NBUF - 1) < NCHUNK)
            def _prefetch():
                fire(ci + b + (NBUF - 1), (b + NBUF - 1) % NBUF)

        return tot

    tot = lax.fori_loop(0, NCHUNK // NBUF, ring_body,
                        jnp.zeros((L,), jnp.float32))

    stage[...] = tot * (-1.0 / B)
    pltpu.sync_copy(stage, shared.at[sid])
    plsc.subcore_barrier()

    @pl.when(sid == 0)
    def _reduce():
        pltpu.sync_copy(shared, gbuf)
        r = gbuf[0, :]
        for t in range(1, NS):
            r = r + gbuf[t, :]
        stage[...] = r
        pltpu.sync_copy(stage, out_hbm.at[cid])


@functools.cache
def _sc_loss():
    return pl.kernel(
        _sc_loss_kernel,
        out_type=jax.ShapeDtypeStruct((NC, L), jnp.float32),
        mesh=plsc.VectorSubcoreMesh(
            core_axis_name="c", subcore_axis_name="s",
            num_cores=NC, num_subcores=NS),
        compiler_params=pltpu.CompilerParams(
            needs_layout_passes=False, use_tc_tiling_on_sc=False),
        scratch_types=[
            pltpu.VMEM((BPW,), jnp.int32),
            pltpu.VMEM((BPW,), jnp.int32),
            pltpu.VMEM((BPW * K,), jnp.int32),
            pltpu.VMEM((C, D), jnp.float32),
            pltpu.VMEM((C, D), jnp.float32),
            pltpu.VMEM((C * K, D), jnp.float32),
            pltpu.VMEM((C, D), jnp.float32),
            pltpu.VMEM((C, D), jnp.float32),
            pltpu.VMEM((C * K, D), jnp.float32),
            pltpu.VMEM((C, D), jnp.float32),
            pltpu.VMEM((C, D), jnp.float32),
            pltpu.VMEM((C * K, D), jnp.float32),
            pltpu.VMEM((C, D), jnp.float32),
            pltpu.VMEM((C, D), jnp.float32),
            pltpu.VMEM((C * K, D), jnp.float32),
            pltpu.VMEM((C, SWP), jnp.float32),
            pltpu.VMEM((L,), jnp.float32),
            pltpu.VMEM((NS, L), jnp.float32),
            pltpu.VMEM_SHARED((NS, L), jnp.float32),
            pltpu.SemaphoreType.DMA,
            pltpu.SemaphoreType.DMA,
            pltpu.SemaphoreType.DMA,
            pltpu.SemaphoreType.DMA,
        ],
    )


def kernel(v_i, v_j, neg_samples, emb, ctx):
    vi = v_i.astype(jnp.int32)
    vj = v_j.astype(jnp.int32)
    vn = neg_samples.astype(jnp.int32).reshape(-1)
    partials = _sc_loss()(vi, vj, vn, emb, ctx)
    return jnp.sum(partials)


# speedup vs baseline: 1.1198x; 1.1198x over previous
import functools

import jax
import jax.numpy as jnp
from jax import lax
from jax.experimental import pallas as pl
from jax.experimental.pallas import tpu as pltpu
from jax.experimental.pallas import tpu_sc as plsc

B = 16384
D = 128
K = 5
L = 16
NC = 2
NS = 16
NW = NC * NS
BPW = B // NW
C = 64
NCHUNK = BPW // C
NV = D // L
SCORES = K + 1
SW = SCORES * L
SWP = SW + 1

_LOG1P = (2.2159764512252877e-07, 0.9999702572822571, -0.4993339478969574,
          0.327511727809906, -0.22396689653396606, 0.13198965787887573,
          -0.053267478942871094, 0.010243828408420086)


def _log_sigmoid(s):
    y = jnp.exp(-jnp.abs(s))
    p = _LOG1P[7] * y + _LOG1P[6]
    for c in _LOG1P[5::-1]:
        p = p * y + c
    return jnp.minimum(s, 0.0) - p


def _sc_loss_kernel(vi_hbm, vj_hbm, vn_hbm, emb_hbm, ctx_hbm, out_hbm,
                    idx_i, idx_j, idx_n,
                    rows_i0, rows_j0, rows_n0,
                    rows_i1, rows_j1, rows_n1,
                    acc, stage, gbuf, shared, sem0, sem1):
    cid = lax.axis_index("c")
    sid = lax.axis_index("s")
    wid = sid * NC + cid
    base = wid * BPW

    pltpu.sync_copy(vi_hbm.at[pl.ds(base, BPW)], idx_i)
    pltpu.sync_copy(vj_hbm.at[pl.ds(base, BPW)], idx_j)
    pltpu.sync_copy(vn_hbm.at[pl.ds(base * K, BPW * K)], idx_n)

    bufs = ((rows_i0, rows_j0, rows_n0, sem0),
            (rows_i1, rows_j1, rows_n1, sem1))

    def copies(ci, parity):
        ri, rj, rn, sem = bufs[parity]
        o = ci * C
        cps = [
            (emb_hbm.at[idx_i.at[pl.ds(o, C)]], ri, sem),
            (ctx_hbm.at[idx_j.at[pl.ds(o, C)]], rj, sem),
        ]
        for s, n in ((0, 128), (128, 128), (256, 64)):
            cps.append((ctx_hbm.at[idx_n.at[pl.ds(o * K + s, n)]],
                        rn.at[pl.ds(s, n), :], sem))
        return cps

    def fire(ci, parity):
        for src, dst, sem in copies(ci, parity):
            pltpu.async_copy(src, dst, sem)

    def drain(ci, parity):
        for src, dst, sem in copies(ci, parity):
            pltpu.make_async_copy(src, dst, sem).wait()

    iota16 = lax.iota(jnp.int32, L)
    zeros16 = iota16 * 0

    def compute(parity, tot):
        ri, rj, rn, _ = bufs[parity]

        def tree_dot(u, ref, row):
            ps = [u[l] * ref[row, pl.ds(L * l, L)] for l in range(NV)]
            while len(ps) > 1:
                ps = [a + b for a, b in zip(ps[::2], ps[1::2])]
            return ps[0]

        @plsc.parallel_loop(0, C, unroll=4)
        def _stage_a(e):
            u = [ri[e, pl.ds(L * l, L)] for l in range(NV)]
            acc[e, pl.ds(0, L)] = tree_dot(u, rj, e)
            for k in range(K):
                acc[e, pl.ds(L * (k + 1), L)] = tree_dot(u, rn, K * e + k)

        @plsc.parallel_loop(0, C // L, unroll=2, carry=tot)
        def tot(g, tot2):
            rowidx = iota16 + g * L
            for j in range(SCORES):
                gs = [plsc.load_gather(acc, [rowidx, zeros16 + (L * j + l)])
                      for l in range(L)]
                while len(gs) > 1:
                    gs = [a + b for a, b in zip(gs[::2], gs[1::2])]
                tot2 = tot2 + _log_sigmoid(gs[0] if j == 0 else -gs[0])
            return tot2
        return tot

    fire(0, 0)
    fire(1, 1)

    def pair_body(p, tot):
        ci = 2 * p
        drain(ci, 0)
        tot = compute(0, tot)

        @pl.when(ci + 2 < NCHUNK)
        def _prefetch0():
            fire(ci + 2, 0)

        drain(ci + 1, 1)
        tot = compute(1, tot)

        @pl.when(ci + 3 < NCHUNK)
        def _prefetch1():
            fire(ci + 3, 1)

        return tot

    tot = lax.fori_loop(0, NCHUNK // 2, pair_body,
                        jnp.zeros((L,), jnp.float32))

    stage[...] = tot * (-1.0 / B)
    pltpu.sync_copy(stage, shared.at[sid])
    plsc.subcore_barrier()

    @pl.when(sid == 0)
    def _reduce():
        pltpu.sync_copy(shared, gbuf)
        r = gbuf[0, :]
        for t in range(1, NS):
            r = r + gbuf[t, :]
        stage[...] = r
        pltpu.sync_copy(stage, out_hbm.at[cid])


@functools.cache
def _sc_loss():
    return pl.kernel(
        _sc_loss_kernel,
        out_type=jax.ShapeDtypeStruct((NC, L), jnp.float32),
        mesh=plsc.VectorSubcoreMesh(
            core_axis_name="c", subcore_axis_name="s",
            num_cores=NC, num_subcores=NS),
        compiler_params=pltpu.CompilerParams(
            needs_layout_passes=False, use_tc_tiling_on_sc=False),
        scratch_types=[
            pltpu.VMEM((BPW,), jnp.int32),
            pltpu.VMEM((BPW,), jnp.int32),
            pltpu.VMEM((BPW * K,), jnp.int32),
            pltpu.VMEM((C, D), jnp.float32),
            pltpu.VMEM((C, D), jnp.float32),
            pltpu.VMEM((C * K, D), jnp.float32),
            pltpu.VMEM((C, D), jnp.float32),
            pltpu.VMEM((C, D), jnp.float32),
            pltpu.VMEM((C * K, D), jnp.float32),
            pltpu.VMEM((C, SWP), jnp.float32),
            pltpu.VMEM((L,), jnp.float32),
            pltpu.VMEM((NS, L), jnp.float32),
            pltpu.VMEM_SHARED((NS, L), jnp.float32),
            pltpu.SemaphoreType.DMA,
            pltpu.SemaphoreType.DMA,
        ],
    )


def kernel(v_i, v_j, neg_samples, emb, ctx):
    vi = v_i.astype(jnp.int32)
    vj = v_j.astype(jnp.int32)
    vn = neg_samples.astype(jnp.int32).reshape(-1)
    partials = _sc_loss()(vi, vj, vn, emb, ctx)
    return jnp.sum(partials)
